# Initial kernel scaffold; baseline (speedup 1.0000x reference)
import jax
import jax.numpy as jnp
from jax.experimental import pallas as pl


def kernel(point_features, coords, W0, b0, W1, b1, W2, b2):
    raise NotImplementedError("write your pallas kernel here")


# speedup vs baseline: 2.9306x; 2.9306x over previous
import functools

import jax
import jax.numpy as jnp
from jax.experimental import pallas as pl

_K = 32
_BLK = 128
_HIGH = jax.lax.Precision.HIGHEST


def _dot(a, b, dims):
    return jax.lax.dot_general(a, b, (dims, ((), ())), precision=_HIGH,
                               preferred_element_type=jnp.float32)


def _body(feats_ref, ct_ref, cb_ref, W0_ref, W1_ref, W2_ref,
          b0_ref, b1_ref, b2_ref, out_ref, *, n, blk, k):
    cb = cb_ref[...]

    def comp(c):
        row = cb[:, c].reshape(blk, 1)
        col = ct_ref[c, :].reshape(1, n)
        diff = row - col
        return diff * diff

    d = (comp(0) + comp(1)) + comp(2)

    iota = jax.lax.broadcasted_iota(jnp.int32, (blk, n), 1)
    inf = jnp.float32(jnp.inf)

    def extract(d, sel):
        m = jnp.min(d, axis=1, keepdims=True)
        cand = d == m
        jstar = jnp.min(jnp.where(cand, iota, n), axis=1, keepdims=True)
        mask = iota == jstar
        d = jnp.where(mask, inf, d)
        sel = sel + mask.astype(jnp.float32)
        return d, sel, mask

    sel = jnp.zeros((blk, n), jnp.float32)
    d, sel, mask0 = extract(d, sel)
    sel_first = mask0.astype(jnp.float32)

    def step(_, carry):
        d, sel = carry
        d, sel, _ = extract(d, sel)
        return (d, sel)

    d, sel = jax.lax.fori_loop(1, k, step, (d, sel))

    feats = feats_ref[...]
    ct = ct_ref[...]
    g_feat = _dot(sel, feats, ((1,), (0,)))
    g_coord = _dot(sel, ct, ((1,), (1,)))
    c0 = _dot(sel_first, ct, ((1,), (1,)))
    x_coord = g_coord - jnp.float32(k) * c0

    W0 = W0_ref[...]
    W1 = W1_ref[...]
    W2 = W2_ref[...]
    W10 = _dot(W1, W0, ((1,), (0,)))
    Weff = _dot(W2, W10, ((1,), (0,)))
    Wf = Weff[:, :64]
    Wc = Weff[:, 64:67]

    t = _dot(b0_ref[...], W1, ((1,), (1,))) + b1_ref[...]
    beff = _dot(t, W2, ((1,), (1,))) + b2_ref[...]

    out = (_dot(g_feat, Wf, ((1,), (1,)))
           + _dot(x_coord, Wc, ((1,), (1,)))
           + jnp.float32(k) * beff)
    out_ref[...] = out


def _run_one(feats, coords, W0, b0, W1, b1, W2, b2):
    n, cin = feats.shape
    blk = _BLK if n % _BLK == 0 else n
    coords_t = coords.T
    body = functools.partial(_body, n=n, blk=blk, k=_K)
    out = pl.pallas_call(
        body,
        grid=(n // blk,),
        in_specs=[
            pl.BlockSpec((n, cin), lambda i: (0, 0)),
            pl.BlockSpec((3, n), lambda i: (0, 0)),
            pl.BlockSpec((blk, 3), lambda i: (i, 0)),
            pl.BlockSpec(W0.shape, lambda i: (0, 0)),
            pl.BlockSpec(W1.shape, lambda i: (0, 0)),
            pl.BlockSpec(W2.shape, lambda i: (0, 0)),
            pl.BlockSpec((1, W0.shape[0]), lambda i: (0, 0)),
            pl.BlockSpec((1, W1.shape[0]), lambda i: (0, 0)),
            pl.BlockSpec((1, W2.shape[0]), lambda i: (0, 0)),
        ],
        out_specs=pl.BlockSpec((blk, W2.shape[0]), lambda i: (i, 0)),
        out_shape=jax.ShapeDtypeStruct((n, W2.shape[0]), jnp.float32),
    )(feats, coords_t, coords, W0, W1, W2,
      b0.reshape(1, -1), b1.reshape(1, -1), b2.reshape(1, -1))
    return out


def kernel(point_features, coords, W0, b0, W1, b1, W2, b2):
    outs = [
        _run_one(point_features[b], coords[b], W0, b0, W1, b1, W2, b2)
        for b in range(point_features.shape[0])
    ]
    return jnp.stack(outs, axis=0)


# speedup vs baseline: 10.3448x; 3.5300x over previous
import functools

import jax
import jax.numpy as jnp
from jax.experimental import pallas as pl

_K = 32
_BLK = 256
_HIGH = jax.lax.Precision.HIGHEST
_INF_BITS = 0x7F800000


def _dot(a, b, dims):
    return jax.lax.dot_general(a, b, (dims, ((), ())), precision=_HIGH,
                               preferred_element_type=jnp.float32)


def _body(feats_ref, ct_ref, cb_ref, W0_ref, W1_ref, W2_ref,
          b0_ref, b1_ref, b2_ref, out_ref, *, n, blk, k):
    cb = cb_ref[...]

    def comp(c):
        row = cb[:, c].reshape(blk, 1)
        col = ct_ref[c, :].reshape(1, n)
        diff = row - col
        return diff * diff

    d = (comp(0) + comp(1)) + comp(2)
    ki = jax.lax.bitcast_convert_type(d, jnp.int32)

    iota = jax.lax.broadcasted_iota(jnp.int32, (blk, n), 1)

    ct = ct_ref[...]
    m0 = jnp.min(ki, axis=1, keepdims=True)
    j0 = jnp.min(jnp.where(ki == m0, iota, n), axis=1, keepdims=True)
    sel_first = (iota == j0).astype(jnp.float32)
    c0 = _dot(sel_first, ct, ((1,), (1,)))

    lo = jnp.zeros((blk, 1), jnp.int32)
    hi = jnp.full((blk, 1), _INF_BITS, jnp.int32)

    def bstep(_, carry):
        lo, hi = carry
        mid = lo + ((hi - lo) >> 1)
        cnt = jnp.sum((ki < mid).astype(jnp.int32), axis=1, keepdims=True)
        pred = cnt <= (k - 1)
        return (jnp.where(pred, mid, lo), jnp.where(pred, hi, mid))

    lo, hi = jax.lax.fori_loop(0, 31, bstep, (lo, hi))
    vk = lo

    sel = (ki < vk).astype(jnp.float32)
    deficit = k - jnp.sum(sel, axis=1, keepdims=True).astype(jnp.int32)

    def fcond(st):
        _, _, deficit = st
        return jnp.max(deficit) > 0

    def fbody(st):
        sel, lastpick, deficit = st
        active = deficit > 0
        cand = (ki == vk) & (iota > lastpick)
        jidx = jnp.min(jnp.where(cand, iota, n), axis=1, keepdims=True)
        add = active & (iota == jidx)
        sel = sel + add.astype(jnp.float32)
        lastpick = jnp.where(active, jidx, lastpick)
        deficit = deficit - active.astype(jnp.int32)
        return (sel, lastpick, deficit)

    lastpick = jnp.full((blk, 1), -1, jnp.int32)
    sel, _, _ = jax.lax.while_loop(fcond, fbody, (sel, lastpick, deficit))

    feats = feats_ref[...]
    g_feat = _dot(sel, feats, ((1,), (0,)))
    g_coord = _dot(sel, ct, ((1,), (1,)))
    x_coord = g_coord - jnp.float32(k) * c0

    W0 = W0_ref[...]
    W1 = W1_ref[...]
    W2 = W2_ref[...]
    W10 = _dot(W1, W0, ((1,), (0,)))
    Weff = _dot(W2, W10, ((1,), (0,)))
    Wf = Weff[:, :64]
    Wc = Weff[:, 64:67]

    t = _dot(b0_ref[...], W1, ((1,), (1,))) + b1_ref[...]
    beff = _dot(t, W2, ((1,), (1,))) + b2_ref[...]

    out = (_dot(g_feat, Wf, ((1,), (1,)))
           + _dot(x_coord, Wc, ((1,), (1,)))
           + jnp.float32(k) * beff)
    out_ref[...] = out


def _run_one(feats, coords, W0, b0, W1, b1, W2, b2):
    n, cin = feats.shape
    blk = _BLK if n % _BLK == 0 else n
    coords_t = coords.T
    body = functools.partial(_body, n=n, blk=blk, k=_K)
    out = pl.pallas_call(
        body,
        grid=(n // blk,),
        in_specs=[
            pl.BlockSpec((n, cin), lambda i: (0, 0)),
            pl.BlockSpec((3, n), lambda i: (0, 0)),
            pl.BlockSpec((blk, 3), lambda i: (i, 0)),
            pl.BlockSpec(W0.shape, lambda i: (0, 0)),
            pl.BlockSpec(W1.shape, lambda i: (0, 0)),
            pl.BlockSpec(W2.shape, lambda i: (0, 0)),
            pl.BlockSpec((1, W0.shape[0]), lambda i: (0, 0)),
            pl.BlockSpec((1, W1.shape[0]), lambda i: (0, 0)),
            pl.BlockSpec((1, W2.shape[0]), lambda i: (0, 0)),
        ],
        out_specs=pl.BlockSpec((blk, W2.shape[0]), lambda i: (i, 0)),
        out_shape=jax.ShapeDtypeStruct((n, W2.shape[0]), jnp.float32),
    )(feats, coords_t, coords, W0, W1, W2,
      b0.reshape(1, -1), b1.reshape(1, -1), b2.reshape(1, -1))
    return out


def kernel(point_features, coords, W0, b0, W1, b1, W2, b2):
    outs = [
        _run_one(point_features[b], coords[b], W0, b0, W1, b1, W2, b2)
        for b in range(point_features.shape[0])
    ]
    return jnp.stack(outs, axis=0)


# speedup vs baseline: 14.2557x; 1.3781x over previous
import functools

import jax
import jax.numpy as jnp
from jax.experimental import pallas as pl

_K = 32
_BLK = 256
_HIGH = jax.lax.Precision.HIGHEST
_INF_BITS = 0x7F800000


def _dot(a, b, dims):
    return jax.lax.dot_general(a, b, (dims, ((), ())), precision=_HIGH,
                               preferred_element_type=jnp.float32)


def _bdot(a, b):
    return jax.lax.dot_general(a, b, ((((1,), (0,))), ((), ())),
                               preferred_element_type=jnp.float32)


def _body(rhs_ref, ct_ref, cb_ref, W0_ref, W1_ref, W2_ref,
          b0_ref, b1_ref, b2_ref, out_ref, *, n, blk, k, cin):
    cb = cb_ref[...]

    def comp(c):
        row = cb[:, c].reshape(blk, 1)
        col = ct_ref[c, :].reshape(1, n)
        diff = row - col
        return diff * diff

    d = (comp(0) + comp(1)) + comp(2)
    ki = jax.lax.bitcast_convert_type(d, jnp.int32)

    iota = jax.lax.broadcasted_iota(jnp.int32, (blk, n), 1)

    m0 = jnp.min(ki, axis=1, keepdims=True)
    j0 = jnp.min(jnp.where(ki == m0, iota, n), axis=1, keepdims=True)
    sel_first = (iota == j0).astype(jnp.bfloat16)

    lo = jnp.zeros((blk, 1), jnp.int32)
    hi = jnp.full((blk, 1), _INF_BITS, jnp.int32)

    def bstep(_, carry):
        lo, hi = carry
        mid = lo + ((hi - lo) >> 1)
        negcnt = jnp.sum((ki - mid) >> 31, axis=1, keepdims=True)
        pred = negcnt >= -(k - 1)
        return (jnp.where(pred, mid, lo), jnp.where(pred, hi, mid))

    lo, hi = jax.lax.fori_loop(0, 31, bstep, (lo, hi))
    vk = lo

    sel = (ki < vk).astype(jnp.bfloat16)
    deficit = k + jnp.sum((ki - vk) >> 31, axis=1, keepdims=True)

    def fcond(st):
        _, _, deficit = st
        return jnp.max(deficit) > 0

    def fbody(st):
        sel, lastpick, deficit = st
        active = deficit > 0
        cand = (ki == vk) & (iota > lastpick)
        jidx = jnp.min(jnp.where(cand, iota, n), axis=1, keepdims=True)
        add = active & (iota == jidx)
        sel = sel + add.astype(jnp.bfloat16)
        lastpick = jnp.where(active, jidx, lastpick)
        deficit = deficit - active.astype(jnp.int32)
        return (sel, lastpick, deficit)

    lastpick = jnp.full((blk, 1), -1, jnp.int32)
    sel, _, _ = jax.lax.while_loop(fcond, fbody, (sel, lastpick, deficit))

    rhs = rhs_ref[...]
    gsum = _bdot(sel, rhs)
    gfirst = _bdot(sel_first, rhs)
    c = cin + 3
    g_feat = gsum[:, :cin] + gsum[:, c:c + cin]
    g_coord = gsum[:, cin:c] + gsum[:, c + cin:2 * c]
    c0 = gfirst[:, cin:c] + gfirst[:, c + cin:2 * c]
    x_coord = g_coord - jnp.float32(k) * c0

    W0 = W0_ref[...]
    W1 = W1_ref[...]
    W2 = W2_ref[...]
    W10 = _dot(W1, W0, ((1,), (0,)))
    Weff = _dot(W2, W10, ((1,), (0,)))
    Wf = Weff[:, :64]
    Wc = Weff[:, 64:67]

    t = _dot(b0_ref[...], W1, ((1,), (1,))) + b1_ref[...]
    beff = _dot(t, W2, ((1,), (1,))) + b2_ref[...]

    out = (_dot(g_feat, Wf, ((1,), (1,)))
           + _dot(x_coord, Wc, ((1,), (1,)))
           + jnp.float32(k) * beff)
    out_ref[...] = out


def _run_one(feats, coords, W0, b0, W1, b1, W2, b2):
    n, cin = feats.shape
    blk = _BLK if n % _BLK == 0 else n
    coords_t = coords.T
    f67 = jnp.concatenate([feats, coords], axis=1)
    f_hi = f67.astype(jnp.bfloat16)
    f_lo = (f67 - f_hi.astype(jnp.float32)).astype(jnp.bfloat16)
    rhs = jnp.concatenate([f_hi, f_lo], axis=1)
    body = functools.partial(_body, n=n, blk=blk, k=_K, cin=cin)
    out = pl.pallas_call(
        body,
        grid=(n // blk,),
        in_specs=[
            pl.BlockSpec(rhs.shape, lambda i: (0, 0)),
            pl.BlockSpec((3, n), lambda i: (0, 0)),
            pl.BlockSpec((blk, 3), lambda i: (i, 0)),
            pl.BlockSpec(W0.shape, lambda i: (0, 0)),
            pl.BlockSpec(W1.shape, lambda i: (0, 0)),
            pl.BlockSpec(W2.shape, lambda i: (0, 0)),
            pl.BlockSpec((1, W0.shape[0]), lambda i: (0, 0)),
            pl.BlockSpec((1, W1.shape[0]), lambda i: (0, 0)),
            pl.BlockSpec((1, W2.shape[0]), lambda i: (0, 0)),
        ],
        out_specs=pl.BlockSpec((blk, W2.shape[0]), lambda i: (i, 0)),
        out_shape=jax.ShapeDtypeStruct((n, W2.shape[0]), jnp.float32),
    )(rhs, coords_t, coords, W0, W1, W2,
      b0.reshape(1, -1), b1.reshape(1, -1), b2.reshape(1, -1))
    return out


def kernel(point_features, coords, W0, b0, W1, b1, W2, b2):
    outs = [
        _run_one(point_features[b], coords[b], W0, b0, W1, b1, W2, b2)
        for b in range(point_features.shape[0])
    ]
    return jnp.stack(outs, axis=0)


# speedup vs baseline: 14.7892x; 1.0374x over previous
import functools

import jax
import jax.numpy as jnp
from jax.experimental import pallas as pl

_K = 32
_BLK = 256
_HIGH = jax.lax.Precision.HIGHEST
_INF_BITS = 0x7F800000


def _dot(a, b, dims):
    return jax.lax.dot_general(a, b, (dims, ((), ())), precision=_HIGH,
                               preferred_element_type=jnp.float32)


def _bdot(a, b):
    return jax.lax.dot_general(a, b, ((((1,), (0,))), ((), ())),
                               preferred_element_type=jnp.float32)


def _body(rhs_ref, ct_ref, cb_ref, W0_ref, W1_ref, W2_ref,
          b0_ref, b1_ref, b2_ref, out_ref, *, n, blk, k, cin):
    cb = cb_ref[...]

    def comp(c):
        row = cb[:, c].reshape(blk, 1)
        col = ct_ref[c, :].reshape(1, n)
        diff = row - col
        return diff * diff

    d = (comp(0) + comp(1)) + comp(2)
    ki = jax.lax.bitcast_convert_type(d, jnp.int32)

    iota = jax.lax.broadcasted_iota(jnp.int32, (blk, n), 1)

    lo = jnp.min(ki, axis=1, keepdims=True)
    hi = jnp.max(ki, axis=1, keepdims=True) + 1
    clo = jnp.zeros((blk, 1), jnp.int32)

    def bcond(carry):
        lo, hi, _ = carry
        return jnp.max(hi - lo) > 1

    def bstep(carry):
        lo, hi, clo = carry
        mid = lo + ((hi - lo) >> 1)
        negcnt = jnp.sum((ki - mid) >> 31, axis=1, keepdims=True)
        pred = negcnt >= -(k - 1)
        return (jnp.where(pred, mid, lo), jnp.where(pred, hi, mid),
                jnp.where(pred, -negcnt, clo))

    lo, hi, clo = jax.lax.while_loop(bcond, bstep, (lo, hi, clo))
    vk = lo

    sel = (ki < vk).astype(jnp.bfloat16)

    jidx = jnp.min(jnp.where(ki == vk, iota, n), axis=1, keepdims=True)
    sel = sel + (iota == jidx).astype(jnp.bfloat16)
    deficit = (k - 1) - clo

    def fcond(st):
        _, _, deficit = st
        return jnp.max(deficit) > 0

    def fbody(st):
        sel, lastpick, deficit = st
        active = deficit > 0
        cand = (ki == vk) & (iota > lastpick)
        jidx = jnp.min(jnp.where(cand, iota, n), axis=1, keepdims=True)
        add = active & (iota == jidx)
        sel = sel + add.astype(jnp.bfloat16)
        lastpick = jnp.where(active, jidx, lastpick)
        deficit = deficit - active.astype(jnp.int32)
        return (sel, lastpick, deficit)

    sel, _, _ = jax.lax.while_loop(fcond, fbody, (sel, jidx, deficit))

    rhs = rhs_ref[...]
    gsum = _bdot(sel, rhs)
    c = cin + 3
    g_feat = gsum[:, :cin] + gsum[:, c:c + cin]
    g_coord = gsum[:, cin:c] + gsum[:, c + cin:2 * c]
    x_coord = g_coord - jnp.float32(k) * cb

    W0 = W0_ref[...]
    W1 = W1_ref[...]
    W2 = W2_ref[...]
    W10 = _dot(W1, W0, ((1,), (0,)))
    Weff = _dot(W2, W10, ((1,), (0,)))
    Wf = Weff[:, :64]
    Wc = Weff[:, 64:67]

    t = _dot(b0_ref[...], W1, ((1,), (1,))) + b1_ref[...]
    beff = _dot(t, W2, ((1,), (1,))) + b2_ref[...]

    out = (_dot(g_feat, Wf, ((1,), (1,)))
           + _dot(x_coord, Wc, ((1,), (1,)))
           + jnp.float32(k) * beff)
    out_ref[...] = out


def _run_one(feats, coords, W0, b0, W1, b1, W2, b2):
    n, cin = feats.shape
    blk = _BLK if n % _BLK == 0 else n
    coords_t = coords.T
    f67 = jnp.concatenate([feats, coords], axis=1)
    f_hi = f67.astype(jnp.bfloat16)
    f_lo = (f67 - f_hi.astype(jnp.float32)).astype(jnp.bfloat16)
    rhs = jnp.concatenate([f_hi, f_lo], axis=1)
    body = functools.partial(_body, n=n, blk=blk, k=_K, cin=cin)
    out = pl.pallas_call(
        body,
        grid=(n // blk,),
        in_specs=[
            pl.BlockSpec(rhs.shape, lambda i: (0, 0)),
            pl.BlockSpec((3, n), lambda i: (0, 0)),
            pl.BlockSpec((blk, 3), lambda i: (i, 0)),
            pl.BlockSpec(W0.shape, lambda i: (0, 0)),
            pl.BlockSpec(W1.shape, lambda i: (0, 0)),
            pl.BlockSpec(W2.shape, lambda i: (0, 0)),
            pl.BlockSpec((1, W0.shape[0]), lambda i: (0, 0)),
            pl.BlockSpec((1, W1.shape[0]), lambda i: (0, 0)),
            pl.BlockSpec((1, W2.shape[0]), lambda i: (0, 0)),
        ],
        out_specs=pl.BlockSpec((blk, W2.shape[0]), lambda i: (i, 0)),
        out_shape=jax.ShapeDtypeStruct((n, W2.shape[0]), jnp.float32),
    )(rhs, coords_t, coords, W0, W1, W2,
      b0.reshape(1, -1), b1.reshape(1, -1), b2.reshape(1, -1))
    return out


def kernel(point_features, coords, W0, b0, W1, b1, W2, b2):
    outs = [
        _run_one(point_features[b], coords[b], W0, b0, W1, b1, W2, b2)
        for b in range(point_features.shape[0])
    ]
    return jnp.stack(outs, axis=0)


# speedup vs baseline: 15.2493x; 1.0311x over previous
import functools

import jax
import jax.numpy as jnp
from jax.experimental import pallas as pl

_K = 32
_BLK = 256
_HIGH = jax.lax.Precision.HIGHEST
_INF_BITS = 0x7F800000


def _dot(a, b, dims):
    return jax.lax.dot_general(a, b, (dims, ((), ())), precision=_HIGH,
                               preferred_element_type=jnp.float32)


def _bdot(a, b):
    return jax.lax.dot_general(a, b, ((((1,), (0,))), ((), ())),
                               preferred_element_type=jnp.float32)


def _body(rhs_ref, ct_ref, cb_ref, W0_ref, W1_ref, W2_ref,
          b0_ref, b1_ref, b2_ref, out_ref, *, n, blk, k, cin):
    cb = cb_ref[...]

    def comp(c):
        row = cb[:, c].reshape(blk, 1)
        col = ct_ref[c, :].reshape(1, n)
        diff = row - col
        return diff * diff

    d = (comp(0) + comp(1)) + comp(2)
    ki = jax.lax.bitcast_convert_type(d, jnp.int32)

    iota = jax.lax.broadcasted_iota(jnp.int32, (blk, n), 1)

    lo = jnp.min(ki, axis=1, keepdims=True)
    hi = jnp.max(ki, axis=1, keepdims=True) + 1
    clo = jnp.zeros((blk, 1), jnp.int32)

    def bcond(carry):
        lo, hi, _ = carry
        return jnp.max(hi - lo) > 1

    def bstep(carry):
        lo, hi, clo = carry
        mid = lo + ((hi - lo) >> 1)
        negcnt = jnp.sum((ki - mid) >> 31, axis=1, keepdims=True)
        pred = negcnt >= -(k - 1)
        return (jnp.where(pred, mid, lo), jnp.where(pred, hi, mid),
                jnp.where(pred, -negcnt, clo))

    lo, hi, clo = jax.lax.fori_loop(0, 27, lambda _, c: bstep(c),
                                    (lo, hi, clo))
    lo, hi, clo = jax.lax.while_loop(bcond, bstep, (lo, hi, clo))
    vk = lo

    sel = (ki < vk).astype(jnp.bfloat16)

    jidx = jnp.min(jnp.where(ki == vk, iota, n), axis=1, keepdims=True)
    sel = sel + (iota == jidx).astype(jnp.bfloat16)
    deficit = (k - 1) - clo

    def fcond(st):
        _, _, deficit = st
        return jnp.max(deficit) > 0

    def fbody(st):
        sel, lastpick, deficit = st
        active = deficit > 0
        cand = (ki == vk) & (iota > lastpick)
        jidx = jnp.min(jnp.where(cand, iota, n), axis=1, keepdims=True)
        add = active & (iota == jidx)
        sel = sel + add.astype(jnp.bfloat16)
        lastpick = jnp.where(active, jidx, lastpick)
        deficit = deficit - active.astype(jnp.int32)
        return (sel, lastpick, deficit)

    sel, _, _ = jax.lax.while_loop(fcond, fbody, (sel, jidx, deficit))

    rhs = rhs_ref[...]
    gsum = _bdot(sel, rhs)
    c = cin + 3
    g_feat = gsum[:, :cin] + gsum[:, c:c + cin]
    g_coord = gsum[:, cin:c] + gsum[:, c + cin:2 * c]
    x_coord = g_coord - jnp.float32(k) * cb

    W0 = W0_ref[...]
    W1 = W1_ref[...]
    W2 = W2_ref[...]
    W10 = _dot(W1, W0, ((1,), (0,)))
    Weff = _dot(W2, W10, ((1,), (0,)))
    Wf = Weff[:, :64]
    Wc = Weff[:, 64:67]

    t = _dot(b0_ref[...], W1, ((1,), (1,))) + b1_ref[...]
    beff = _dot(t, W2, ((1,), (1,))) + b2_ref[...]

    out = (_dot(g_feat, Wf, ((1,), (1,)))
           + _dot(x_coord, Wc, ((1,), (1,)))
           + jnp.float32(k) * beff)
    out_ref[...] = out


def _run_one(feats, coords, W0, b0, W1, b1, W2, b2):
    n, cin = feats.shape
    blk = _BLK if n % _BLK == 0 else n
    coords_t = coords.T
    f67 = jnp.concatenate([feats, coords], axis=1)
    f_hi = f67.astype(jnp.bfloat16)
    f_lo = (f67 - f_hi.astype(jnp.float32)).astype(jnp.bfloat16)
    rhs = jnp.concatenate([f_hi, f_lo], axis=1)
    body = functools.partial(_body, n=n, blk=blk, k=_K, cin=cin)
    out = pl.pallas_call(
        body,
        grid=(n // blk,),
        in_specs=[
            pl.BlockSpec(rhs.shape, lambda i: (0, 0)),
            pl.BlockSpec((3, n), lambda i: (0, 0)),
            pl.BlockSpec((blk, 3), lambda i: (i, 0)),
            pl.BlockSpec(W0.shape, lambda i: (0, 0)),
            pl.BlockSpec(W1.shape, lambda i: (0, 0)),
            pl.BlockSpec(W2.shape, lambda i: (0, 0)),
            pl.BlockSpec((1, W0.shape[0]), lambda i: (0, 0)),
            pl.BlockSpec((1, W1.shape[0]), lambda i: (0, 0)),
            pl.BlockSpec((1, W2.shape[0]), lambda i: (0, 0)),
        ],
        out_specs=pl.BlockSpec((blk, W2.shape[0]), lambda i: (i, 0)),
        out_shape=jax.ShapeDtypeStruct((n, W2.shape[0]), jnp.float32),
    )(rhs, coords_t, coords, W0, W1, W2,
      b0.reshape(1, -1), b1.reshape(1, -1), b2.reshape(1, -1))
    return out


def kernel(point_features, coords, W0, b0, W1, b1, W2, b2):
    outs = [
        _run_one(point_features[b], coords[b], W0, b0, W1, b1, W2, b2)
        for b in range(point_features.shape[0])
    ]
    return jnp.stack(outs, axis=0)


# speedup vs baseline: 16.4130x; 1.0763x over previous
import functools

import jax
import jax.numpy as jnp
from jax.experimental import pallas as pl

_K = 32
_BLK = 256
_HIGH = jax.lax.Precision.HIGHEST
_INF_BITS = 0x7F800000


def _dot(a, b, dims):
    return jax.lax.dot_general(a, b, (dims, ((), ())), precision=_HIGH,
                               preferred_element_type=jnp.float32)


def _bdot(a, b):
    return jax.lax.dot_general(a, b, ((((1,), (0,))), ((), ())),
                               preferred_element_type=jnp.float32)


def _body(rhs_ref, ct_ref, cb_ref, W0_ref, W1_ref, W2_ref,
          b0_ref, b1_ref, b2_ref, out_ref, *, n, blk, k, cin):
    cb = cb_ref[...]

    def comp(c):
        row = cb[:, c].reshape(blk, 1)
        col = ct_ref[c, :].reshape(1, n)
        diff = row - col
        return diff * diff

    d = (comp(0) + comp(1)) + comp(2)
    ki = jax.lax.bitcast_convert_type(d, jnp.int32)

    lo = jnp.min(ki, axis=1, keepdims=True)
    hi = jnp.max(ki, axis=1, keepdims=True) + 1

    def bcond(carry):
        lo, hi = carry
        return jnp.max(hi - lo) > 1

    def bstep(carry):
        lo, hi = carry
        mid = lo + ((hi - lo) >> 1)
        negcnt = jnp.sum((ki - mid) >> 31, axis=1, keepdims=True)
        pred = negcnt >= -(k - 1)
        return (jnp.where(pred, mid, lo), jnp.where(pred, hi, mid))

    lo, hi = jax.lax.fori_loop(0, 27, lambda _, c: bstep(c), (lo, hi))
    lo, hi = jax.lax.while_loop(bcond, bstep, (lo, hi))
    vk = lo

    sel = (ki <= vk).astype(jnp.bfloat16)
    excess = -jnp.sum((ki - (vk + 1)) >> 31, axis=1, keepdims=True) - k

    def fcond(st):
        _, _, excess = st
        return jnp.max(excess) > 0

    def fbody(st):
        sel, lastrem, excess = st
        iota = jax.lax.broadcasted_iota(jnp.int32, (blk, n), 1)
        active = excess > 0
        cand = (ki == vk) & (iota < lastrem)
        jmax = jnp.max(jnp.where(cand, iota, -1), axis=1, keepdims=True)
        rem = active & (iota == jmax)
        sel = sel - rem.astype(jnp.bfloat16)
        lastrem = jnp.where(active, jmax, lastrem)
        excess = excess - active.astype(jnp.int32)
        return (sel, lastrem, excess)

    lastrem = jnp.full((blk, 1), n, jnp.int32)
    sel, _, _ = jax.lax.while_loop(fcond, fbody, (sel, lastrem, excess))

    rhs = rhs_ref[...]
    gsum = _bdot(sel, rhs)
    c = cin + 3
    g_feat = gsum[:, :cin] + gsum[:, c:c + cin]
    g_coord = gsum[:, cin:c] + gsum[:, c + cin:2 * c]
    x_coord = g_coord - jnp.float32(k) * cb

    W0 = W0_ref[...]
    W1 = W1_ref[...]
    W2 = W2_ref[...]
    W10 = _dot(W1, W0, ((1,), (0,)))
    Weff = _dot(W2, W10, ((1,), (0,)))
    Wf = Weff[:, :64]
    Wc = Weff[:, 64:67]

    t = _dot(b0_ref[...], W1, ((1,), (1,))) + b1_ref[...]
    beff = _dot(t, W2, ((1,), (1,))) + b2_ref[...]

    out = (_dot(g_feat, Wf, ((1,), (1,)))
           + _dot(x_coord, Wc, ((1,), (1,)))
           + jnp.float32(k) * beff)
    out_ref[...] = out


def _run_one(feats, coords, W0, b0, W1, b1, W2, b2):
    n, cin = feats.shape
    blk = _BLK if n % _BLK == 0 else n
    coords_t = coords.T
    f67 = jnp.concatenate([feats, coords], axis=1)
    f_hi = f67.astype(jnp.bfloat16)
    f_lo = (f67 - f_hi.astype(jnp.float32)).astype(jnp.bfloat16)
    rhs = jnp.concatenate([f_hi, f_lo], axis=1)
    body = functools.partial(_body, n=n, blk=blk, k=_K, cin=cin)
    out = pl.pallas_call(
        body,
        grid=(n // blk,),
        in_specs=[
            pl.BlockSpec(rhs.shape, lambda i: (0, 0)),
            pl.BlockSpec((3, n), lambda i: (0, 0)),
            pl.BlockSpec((blk, 3), lambda i: (i, 0)),
            pl.BlockSpec(W0.shape, lambda i: (0, 0)),
            pl.BlockSpec(W1.shape, lambda i: (0, 0)),
            pl.BlockSpec(W2.shape, lambda i: (0, 0)),
            pl.BlockSpec((1, W0.shape[0]), lambda i: (0, 0)),
            pl.BlockSpec((1, W1.shape[0]), lambda i: (0, 0)),
            pl.BlockSpec((1, W2.shape[0]), lambda i: (0, 0)),
        ],
        out_specs=pl.BlockSpec((blk, W2.shape[0]), lambda i: (i, 0)),
        out_shape=jax.ShapeDtypeStruct((n, W2.shape[0]), jnp.float32),
    )(rhs, coords_t, coords, W0, W1, W2,
      b0.reshape(1, -1), b1.reshape(1, -1), b2.reshape(1, -1))
    return out


def kernel(point_features, coords, W0, b0, W1, b1, W2, b2):
    outs = [
        _run_one(point_features[b], coords[b], W0, b0, W1, b1, W2, b2)
        for b in range(point_features.shape[0])
    ]
    return jnp.stack(outs, axis=0)
